# baseline (speedup 1.0000x reference)
import functools

import jax
import jax.numpy as jnp
from jax import lax
from jax.experimental import pallas as pl
from jax.experimental.pallas import tpu as pltpu
from jax.experimental.pallas import tpu_sc as plsc

NTA = 16384
O = 2048
NMAX = 2048
E = 8

B = 256
PAD_N = NTA + E * B
NBLK = PAD_N // B
TN = 512
NT = NMAX // TN

_NC, _NS = 2, 16
_NW = _NC * _NS


def _gather_body(n_rows, chunk, table_hbm, idx_hbm, out_hbm, idx_v,
                 rows0, rows1, sem0, sem1):
    b_per_w = n_rows // _NW
    n_pairs = b_per_w // (2 * chunk)
    wid = lax.axis_index("s") * _NC + lax.axis_index("c")
    base = wid * b_per_w
    pltpu.sync_copy(idx_hbm.at[pl.ds(base, b_per_w)], idx_v)

    def body(k, _):
        i0 = 2 * k * chunk
        i1 = i0 + chunk
        c0 = pltpu.async_copy(table_hbm.at[idx_v.at[pl.ds(i0, chunk)]], rows0, sem0)
        c1 = pltpu.async_copy(table_hbm.at[idx_v.at[pl.ds(i1, chunk)]], rows1, sem1)
        c0.wait()
        pltpu.sync_copy(rows0, out_hbm.at[pl.ds(base + i0, chunk)])
        c1.wait()
        pltpu.sync_copy(rows1, out_hbm.at[pl.ds(base + i1, chunk)])
        return 0

    lax.fori_loop(0, n_pairs, body, 0)


def _sc_row_gather(table, idx, n_rows, chunk=16):
    mesh = plsc.VectorSubcoreMesh(core_axis_name="c", subcore_axis_name="s")
    return pl.kernel(
        functools.partial(_gather_body, n_rows, chunk),
        out_type=jax.ShapeDtypeStruct((n_rows, O), jnp.float32),
        mesh=mesh,
        scratch_types=[
            pltpu.VMEM((n_rows // _NW,), jnp.int32),
            pltpu.VMEM((chunk, O), jnp.float32),
            pltpu.VMEM((chunk, O), jnp.float32),
            pltpu.SemaphoreType.DMA,
            pltpu.SemaphoreType.DMA,
        ],
    )(table, idx)


def _mm_body(expert_ref, x_ref, w_ref, b_ref, o_ref):
    x16 = x_ref[...].astype(jnp.bfloat16)
    o_ref[...] = jnp.dot(x16, w_ref[0],
                         preferred_element_type=jnp.float32) + b_ref[0]


def _expert_matmul(rho_s, W, b, block_expert):
    grid_spec = pltpu.PrefetchScalarGridSpec(
        num_scalar_prefetch=1,
        grid=(NBLK,),
        in_specs=[
            pl.BlockSpec((B, O), lambda i, e_ref: (i, 0)),
            pl.BlockSpec((1, O, NMAX), lambda i, e_ref: (e_ref[i], 0, 0)),
            pl.BlockSpec((1, 1, NMAX), lambda i, e_ref: (e_ref[i], 0, 0)),
        ],
        out_specs=pl.BlockSpec((B, NMAX), lambda i, e_ref: (i, 0)),
    )
    return pl.pallas_call(
        _mm_body,
        grid_spec=grid_spec,
        out_shape=jax.ShapeDtypeStruct((PAD_N, NMAX), jnp.float32),
        compiler_params=pltpu.CompilerParams(
            dimension_semantics=("arbitrary",)),
    )(block_expert, rho_s, W, b.reshape(E, 1, NMAX))


def kernel(rho, symbols, W, b):
    sym = symbols.astype(jnp.int32)

    i_arr = jnp.arange(NTA, dtype=jnp.int32)
    scramble = (i_arr * 40503) & (NTA - 1)
    sidx = jnp.argsort(sym * NTA + scramble).astype(jnp.int32)
    inv = jnp.argsort(sidx).astype(jnp.int32)
    e_ids = jnp.arange(E, dtype=jnp.int32)
    counts = (sym[:, None] == e_ids[None, :]).sum(0).astype(jnp.int32)
    starts = jnp.cumsum(counts) - counts
    padded_counts = ((counts + B - 1) // B) * B
    pcum = jnp.cumsum(padded_counts)
    pstarts = pcum - padded_counts

    q_arr = jnp.arange(PAD_N, dtype=jnp.int32)
    e_q = jnp.minimum((q_arr[:, None] >= pcum[None, :]).sum(1), E - 1)
    r_q = q_arr - pstarts[e_q]
    valid = r_q < counts[e_q]
    src_p = jnp.clip(starts[e_q] + r_q, 0, NTA - 1)
    gidx = jnp.where(valid, sidx[src_p], q_arr & (NTA - 1))
    pos = (pstarts[sym] + (inv - starts[sym])).astype(jnp.int32)
    nb = jnp.arange(NBLK, dtype=jnp.int32) * B
    block_expert = jnp.minimum((nb[:, None] >= pcum[None, :]).sum(1), E - 1
                               ).astype(jnp.int32)

    rho_s = _sc_row_gather(rho, gidx, PAD_N)
    y_s = _expert_matmul(rho_s, W.astype(jnp.bfloat16), b, block_expert)
    return _sc_row_gather(y_s, pos, NTA)


# speedup vs baseline: 1.0456x; 1.0456x over previous
import functools

import jax
import jax.numpy as jnp
from jax import lax
from jax.experimental import pallas as pl
from jax.experimental.pallas import tpu as pltpu
from jax.experimental.pallas import tpu_sc as plsc

NTA = 16384
O = 2048
NMAX = 2048
E = 8

B = 256
PAD_N = NTA + E * B
NBLK = PAD_N // B
G = 4
SUB = NBLK // G
ROWS_G = SUB * B
CHUNK = 24

_NC, _NS = 2, 16
_NW = _NC * _NS


def _gather_body(table_hbm, idx_hbm, out_hbm, idx_v, rows0, rows1, sem0, sem1):
    rows_per_w = ROWS_G // _NW // CHUNK
    wid = lax.axis_index("s") * _NC + lax.axis_index("c")
    base = wid * (rows_per_w * CHUNK)
    pltpu.sync_copy(idx_hbm.at[wid], idx_v)

    def body(k, _):
        j0 = 2 * k
        j1 = j0 + 1
        c0 = pltpu.async_copy(table_hbm.at[idx_v.at[j0]], rows0, sem0)
        c1 = pltpu.async_copy(table_hbm.at[idx_v.at[j1]], rows1, sem1)
        c0.wait()
        pltpu.sync_copy(rows0, out_hbm.at[pl.ds(base + j0 * CHUNK, CHUNK)])
        c1.wait()
        pltpu.sync_copy(rows1, out_hbm.at[pl.ds(base + j1 * CHUNK, CHUNK)])
        return 0

    lax.fori_loop(0, rows_per_w // 2, body, 0)


def _scatter_body(y_hbm, idx_hbm, out_ref, idx_v, rows0, rows1, sem0, sem1):
    rows_per_w = ROWS_G // _NW // CHUNK
    wid = lax.axis_index("s") * _NC + lax.axis_index("c")
    base = wid * (rows_per_w * CHUNK)
    pltpu.sync_copy(idx_hbm.at[wid], idx_v)

    def body(k, _):
        j0 = 2 * k
        j1 = j0 + 1
        l0 = pltpu.async_copy(y_hbm.at[pl.ds(base + j0 * CHUNK, CHUNK)], rows0, sem0)
        l1 = pltpu.async_copy(y_hbm.at[pl.ds(base + j1 * CHUNK, CHUNK)], rows1, sem1)
        l0.wait()
        s0 = pltpu.async_copy(rows0, out_ref.at[idx_v.at[j0]], sem0)
        l1.wait()
        s1 = pltpu.async_copy(rows1, out_ref.at[idx_v.at[j1]], sem1)
        s0.wait()
        s1.wait()
        return 0

    lax.fori_loop(0, rows_per_w // 2, body, 0)


def _mesh():
    return plsc.VectorSubcoreMesh(core_axis_name="c", subcore_axis_name="s")


_SC_SCRATCH = [
    pltpu.VMEM((ROWS_G // _NW // CHUNK, CHUNK), jnp.int32),
    pltpu.VMEM((CHUNK, O), jnp.float32),
    pltpu.VMEM((CHUNK, O), jnp.float32),
    pltpu.SemaphoreType.DMA,
    pltpu.SemaphoreType.DMA,
]


def _sc_gather(table, idx2d):
    return pl.kernel(
        _gather_body,
        out_type=jax.ShapeDtypeStruct((ROWS_G, O), jnp.float32),
        mesh=_mesh(),
        scratch_types=_SC_SCRATCH,
    )(table, idx2d)


def _sc_scatter(y, idx2d, out_ref):
    pl.kernel(
        _scatter_body,
        out_type=(),
        mesh=_mesh(),
        scratch_types=_SC_SCRATCH,
    )(y, idx2d, out_ref)


def _mm_body(expert_ref, x_ref, w_ref, b_ref, o_ref):
    o_ref[...] = jnp.dot(x_ref[...], w_ref[0]) + b_ref[0]


def _expert_matmul(rho_g, W, b3, block_expert_g):
    grid_spec = pltpu.PrefetchScalarGridSpec(
        num_scalar_prefetch=1,
        grid=(SUB,),
        in_specs=[
            pl.BlockSpec((B, O), lambda i, e_ref: (i, 0)),
            pl.BlockSpec((1, O, NMAX), lambda i, e_ref: (e_ref[i], 0, 0)),
            pl.BlockSpec((1, 1, NMAX), lambda i, e_ref: (e_ref[i], 0, 0)),
        ],
        out_specs=pl.BlockSpec((B, NMAX), lambda i, e_ref: (i, 0)),
    )
    return pl.pallas_call(
        _mm_body,
        grid_spec=grid_spec,
        out_shape=jax.ShapeDtypeStruct((ROWS_G, NMAX), jnp.float32),
        compiler_params=pltpu.CompilerParams(
            dimension_semantics=("arbitrary",)),
    )(block_expert_g, rho_g, W, b3)


def kernel(rho, symbols, W, b):
    sym = symbols.astype(jnp.int32)

    i_arr = jnp.arange(NTA, dtype=jnp.int32)
    scramble = (i_arr * 40503) & (NTA - 1)
    sidx = jnp.argsort(sym * NTA + scramble).astype(jnp.int32)
    e_ids = jnp.arange(E, dtype=jnp.int32)
    counts = (sym[:, None] == e_ids[None, :]).sum(0).astype(jnp.int32)
    starts = jnp.cumsum(counts) - counts
    padded_counts = ((counts + B - 1) // B) * B
    pcum = jnp.cumsum(padded_counts)
    pstarts = pcum - padded_counts
    total = pcum[-1]

    q_arr = jnp.arange(PAD_N, dtype=jnp.int32)
    qq = jnp.where(q_arr < total, q_arr, q_arr - total)
    e_q = jnp.minimum((qq[:, None] >= pcum[None, :]).sum(1), E - 1)
    r_q = qq - pstarts[e_q]
    src_p = starts[e_q] + r_q % jnp.maximum(counts[e_q], 1)
    gidx = sidx[src_p]
    gidx4d = gidx.reshape(G, _NW, ROWS_G // _NW // CHUNK, CHUNK)

    jb = jnp.arange(NBLK, dtype=jnp.int32) * B
    jb = jnp.where(jb < total, jb, jb - total)
    block_expert = jnp.minimum((jb[:, None] >= pcum[None, :]).sum(1), E - 1
                               ).astype(jnp.int32)

    b3 = b.reshape(E, 1, NMAX)
    out_ref = jax.new_ref(jnp.zeros((NTA, NMAX), jnp.float32))
    for g in range(G):
        idx_g = gidx4d[g]
        rho_g = _sc_gather(rho, idx_g)
        y_g = _expert_matmul(rho_g, W, b3, block_expert[g * SUB:(g + 1) * SUB])
        _sc_scatter(y_g, idx_g, out_ref)
    return out_ref[...]


# speedup vs baseline: 1.1120x; 1.0635x over previous
import functools

import jax
import jax.numpy as jnp
from jax import lax
from jax.experimental import pallas as pl
from jax.experimental.pallas import tpu as pltpu
from jax.experimental.pallas import tpu_sc as plsc

NTA = 16384
O = 2048
NMAX = 2048
E = 8

B = 256
PAD_N = NTA + E * B
NBLK = PAD_N // B
G = 4
SUB = NBLK // G
ROWS_G = SUB * B
CHUNK = 24

_NC, _NS = 2, 16
_NW = _NC * _NS


def _sc_move_body(gather, lin_hbm, idx_hbm, rnd_hbm, idx_v, rows0, rows1,
                  sem0, sem1):
    rows_per_w = ROWS_G // _NW // CHUNK
    wid = lax.axis_index("s") * _NC + lax.axis_index("c")
    base = wid * (rows_per_w * CHUNK)
    pltpu.sync_copy(idx_hbm.at[wid], idx_v)

    bufs = (rows0, rows1)
    sems = (sem0, sem1)
    second = [None, None]
    for j in range(rows_per_w):
        bi = j % 2
        if second[bi] is not None:
            second[bi].wait()
        lin = lin_hbm.at[pl.ds(base + j * CHUNK, CHUNK)]
        rnd = rnd_hbm.at[idx_v.at[j]]
        src, dst = (rnd, lin) if gather else (lin, rnd)
        pltpu.async_copy(src, bufs[bi], sems[bi]).wait()
        second[bi] = pltpu.async_copy(bufs[bi], dst, sems[bi])
    second[0].wait()
    second[1].wait()


def _mesh():
    return plsc.VectorSubcoreMesh(core_axis_name="c", subcore_axis_name="s")


_SC_SCRATCH = [
    pltpu.VMEM((ROWS_G // _NW // CHUNK, CHUNK), jnp.int32),
    pltpu.VMEM((CHUNK, O), jnp.float32),
    pltpu.VMEM((CHUNK, O), jnp.float32),
    pltpu.SemaphoreType.DMA,
    pltpu.SemaphoreType.DMA,
]


def _sc_gather(table, idx3d):
    def body(table_hbm, idx_hbm, out_hbm, *scratch):
        _sc_move_body(True, out_hbm, idx_hbm, table_hbm, *scratch)

    return pl.kernel(
        body,
        out_type=jax.ShapeDtypeStruct((ROWS_G, O), jnp.float32),
        mesh=_mesh(),
        scratch_types=_SC_SCRATCH,
    )(table, idx3d)


def _sc_scatter(y, idx3d, out_ref):
    def body(y_hbm, idx_hbm, o_ref, *scratch):
        _sc_move_body(False, y_hbm, idx_hbm, o_ref, *scratch)

    pl.kernel(
        body,
        out_type=(),
        mesh=_mesh(),
        scratch_types=_SC_SCRATCH,
    )(y, idx3d, out_ref)


def _alloc_body(o_ref):
    pass


def _alloc_out():
    return pl.pallas_call(
        _alloc_body,
        out_shape=jax.ShapeDtypeStruct((NTA, NMAX), jnp.float32),
        out_specs=pl.BlockSpec(memory_space=pltpu.MemorySpace.HBM),
    )()


def _mm_body(expert_ref, x_ref, w_ref, b_ref, o_ref):
    o_ref[...] = jnp.dot(x_ref[...], w_ref[0]) + b_ref[0]


def _expert_matmul(rho_g, W, b3, block_expert_g):
    grid_spec = pltpu.PrefetchScalarGridSpec(
        num_scalar_prefetch=1,
        grid=(SUB,),
        in_specs=[
            pl.BlockSpec((B, O), lambda i, e_ref: (i, 0)),
            pl.BlockSpec((1, O, NMAX), lambda i, e_ref: (e_ref[i], 0, 0)),
            pl.BlockSpec((1, 1, NMAX), lambda i, e_ref: (e_ref[i], 0, 0)),
        ],
        out_specs=pl.BlockSpec((B, NMAX), lambda i, e_ref: (i, 0)),
    )
    return pl.pallas_call(
        _mm_body,
        grid_spec=grid_spec,
        out_shape=jax.ShapeDtypeStruct((ROWS_G, NMAX), jnp.float32),
        compiler_params=pltpu.CompilerParams(
            dimension_semantics=("arbitrary",)),
    )(block_expert_g, rho_g, W, b3)


def kernel(rho, symbols, W, b):
    sym = symbols.astype(jnp.int32)

    i_arr = jnp.arange(NTA, dtype=jnp.int32)
    scramble = (i_arr * 40503) & (NTA - 1)
    sidx = jnp.argsort(sym * NTA + scramble).astype(jnp.int32)
    e_ids = jnp.arange(E, dtype=jnp.int32)
    counts = (sym[:, None] == e_ids[None, :]).sum(0).astype(jnp.int32)
    starts = jnp.cumsum(counts) - counts
    padded_counts = ((counts + B - 1) // B) * B
    pcum = jnp.cumsum(padded_counts)
    pstarts = pcum - padded_counts
    total = pcum[-1]

    q_arr = jnp.arange(PAD_N, dtype=jnp.int32)
    qq = jnp.where(q_arr < total, q_arr, q_arr - total)
    e_q = jnp.minimum((qq[:, None] >= pcum[None, :]).sum(1), E - 1)
    r_q = qq - pstarts[e_q]
    src_p = starts[e_q] + r_q % jnp.maximum(counts[e_q], 1)
    gidx = sidx[src_p]
    gidx4d = gidx.reshape(G, _NW, ROWS_G // _NW // CHUNK, CHUNK)

    jb = jnp.arange(NBLK, dtype=jnp.int32) * B
    jb = jnp.where(jb < total, jb, jb - total)
    block_expert = jnp.minimum((jb[:, None] >= pcum[None, :]).sum(1), E - 1
                               ).astype(jnp.int32)

    b3 = b.reshape(E, 1, NMAX)
    out_ref = jax.new_ref(_alloc_out())
    for g in range(G):
        idx_g = gidx4d[g]
        rho_g = _sc_gather(rho, idx_g)
        y_g = _expert_matmul(rho_g, W, b3, block_expert[g * SUB:(g + 1) * SUB])
        _sc_scatter(y_g, idx_g, out_ref)
    return out_ref[...]
